# baseline (speedup 1.0000x reference)
import jax
import jax.numpy as jnp
from jax.experimental import pallas as pl
from jax.experimental.pallas import tpu as pltpu

N = 4096
M = 4096
BM = 128
GRID = M // BM


def _body(x_ref, y_ref, b_ref, c_ref, iy_ref, xb_ref, yb_ref,
          Q_ref, A_ref, AT_ref,
          res_ref, t1_ref, t2_ref, t3_ref, acc_ref):
    i = pl.program_id(0)

    xT = x_ref[...]
    yT = y_ref[...]
    b_blk = b_ref[...]
    c_blk = c_ref[...]
    iy_blk = iy_ref[...]
    x_blk = xb_ref[...]
    y_blk = yb_ref[...]

    Ax = jnp.sum(A_ref[...] * xT, axis=1, keepdims=True)
    part1 = Ax - b_blk
    part1 = part1 + iy_blk * jnp.maximum(-part1, 0.0)
    s1 = jnp.sum(part1 * part1)

    Qx = jnp.sum(Q_ref[...] * xT, axis=1, keepdims=True)
    ATy = jnp.sum(AT_ref[...] * yT, axis=1, keepdims=True)
    d = Qx + ATy + c_blk
    s2 = jnp.sum(d * d)

    squad = jnp.sum(x_blk * Qx)
    slin = jnp.sum(c_blk * x_blk)
    svio = jnp.sum(b_blk * y_blk)
    sb2 = jnp.sum(b_blk * b_blk)
    sc2 = jnp.sum(c_blk * c_blk)

    parts = (s1, s2, squad, slin, svio, sb2, sc2)

    @pl.when(i == 0)
    def _init():
        for k, v in enumerate(parts):
            acc_ref[k] = v

    @pl.when(i != 0)
    def _accum():
        for k, v in enumerate(parts):
            acc_ref[k] = acc_ref[k] + v

    @pl.when(i == GRID - 1)
    def _fini():
        t1 = jnp.sqrt(acc_ref[0]) / (0.0001 + jnp.sqrt(acc_ref[5]))
        t2 = jnp.sqrt(acc_ref[1]) / (0.0001 + jnp.sqrt(acc_ref[6]))
        t3 = jnp.abs(acc_ref[2] + acc_ref[3] + acc_ref[4])
        t1_ref[0, 0] = t1
        t2_ref[0, 0] = t2
        t3_ref[0, 0] = t3
        res_ref[0, 0] = t1 + t2 + t3


def kernel(Q, A, AT, b, c, x, y, Iy, il, iu, l, u):
    b2 = b[:, None]
    c2 = c[:, None]
    iy2 = Iy[:, None]
    xT = x.T
    yT = y.T

    out_shapes = [jax.ShapeDtypeStruct((1, 1), jnp.float32)] * 4
    full_vec = pl.BlockSpec((1, N), lambda i: (0, 0))
    blk_vec = pl.BlockSpec((BM, 1), lambda i: (i, 0))
    row_blk = pl.BlockSpec((BM, N), lambda i: (i, 0))
    scalar_out = pl.BlockSpec((1, 1), lambda i: (0, 0), memory_space=pltpu.SMEM)

    res, t1, t2, t3 = pl.pallas_call(
        _body,
        grid=(GRID,),
        in_specs=[full_vec, full_vec, blk_vec, blk_vec, blk_vec, blk_vec,
                  blk_vec, row_blk, row_blk, row_blk],
        out_specs=[scalar_out] * 4,
        out_shape=out_shapes,
        scratch_shapes=[pltpu.SMEM((7,), jnp.float32)],
    )(xT, yT, b2, c2, iy2, x, y, Q, A, AT)

    return (res, t1[0, 0], t2[0, 0], t3)


# speedup vs baseline: 1.0180x; 1.0180x over previous
import jax
import jax.numpy as jnp
from jax.experimental import pallas as pl
from jax.experimental.pallas import tpu as pltpu

N = 4096
M = 4096
BM = 256
GRID = M // BM
HC = N // 2


def _body(x_ref, y_ref, b_ref, c_ref, iy_ref, xb_ref, yb_ref,
          Ql_ref, Qr_ref, Al_ref, Ar_ref, ATl_ref, ATr_ref,
          res_ref, t1_ref, t2_ref, t3_ref, acc_ref):
    i = pl.program_id(0)

    xT = x_ref[...]
    yT = y_ref[...]
    xl, xr = xT[:, :HC], xT[:, HC:]
    yl, yr = yT[:, :HC], yT[:, HC:]
    b_blk = b_ref[...]
    c_blk = c_ref[...]
    iy_blk = iy_ref[...]
    x_blk = xb_ref[...]
    y_blk = yb_ref[...]

    def mv(l_ref, r_ref, vl, vr):
        return (jnp.sum(l_ref[...] * vl, axis=1, keepdims=True)
                + jnp.sum(r_ref[...] * vr, axis=1, keepdims=True))

    Ax = mv(Al_ref, Ar_ref, xl, xr)
    part1 = Ax - b_blk
    part1 = part1 + iy_blk * jnp.maximum(-part1, 0.0)
    s1 = jnp.sum(part1 * part1)

    Qx = mv(Ql_ref, Qr_ref, xl, xr)
    ATy = mv(ATl_ref, ATr_ref, yl, yr)
    d = Qx + ATy + c_blk
    s2 = jnp.sum(d * d)

    squad = jnp.sum(x_blk * Qx)
    slin = jnp.sum(c_blk * x_blk)
    svio = jnp.sum(b_blk * y_blk)
    sb2 = jnp.sum(b_blk * b_blk)
    sc2 = jnp.sum(c_blk * c_blk)

    parts = (s1, s2, squad, slin, svio, sb2, sc2)

    @pl.when(i == 0)
    def _init():
        for k, v in enumerate(parts):
            acc_ref[k] = v

    @pl.when(i != 0)
    def _accum():
        for k, v in enumerate(parts):
            acc_ref[k] = acc_ref[k] + v

    @pl.when(i == GRID - 1)
    def _fini():
        t1 = jnp.sqrt(acc_ref[0]) / (0.0001 + jnp.sqrt(acc_ref[5]))
        t2 = jnp.sqrt(acc_ref[1]) / (0.0001 + jnp.sqrt(acc_ref[6]))
        t3 = jnp.abs(acc_ref[2] + acc_ref[3] + acc_ref[4])
        t1_ref[0, 0] = t1
        t2_ref[0, 0] = t2
        t3_ref[0, 0] = t3
        res_ref[0, 0] = t1 + t2 + t3


def kernel(Q, A, AT, b, c, x, y, Iy, il, iu, l, u):
    b2 = b[:, None]
    c2 = c[:, None]
    iy2 = Iy[:, None]
    xT = x.T
    yT = y.T

    out_shapes = [jax.ShapeDtypeStruct((1, 1), jnp.float32)] * 4
    full_vec = pl.BlockSpec((1, N), lambda i: (0, 0))
    blk_vec = pl.BlockSpec((BM, 1), lambda i: (i, 0))
    left_blk = pl.BlockSpec((BM, HC), lambda i: (i, 0))
    right_blk = pl.BlockSpec((BM, HC), lambda i: (i, 1))
    scalar_out = pl.BlockSpec((1, 1), lambda i: (0, 0), memory_space=pltpu.SMEM)

    res, t1, t2, t3 = pl.pallas_call(
        _body,
        grid=(GRID,),
        in_specs=[full_vec, full_vec, blk_vec, blk_vec, blk_vec, blk_vec,
                  blk_vec,
                  left_blk, right_blk, left_blk, right_blk, left_blk,
                  right_blk],
        out_specs=[scalar_out] * 4,
        out_shape=out_shapes,
        scratch_shapes=[pltpu.SMEM((7,), jnp.float32)],
    )(xT, yT, b2, c2, iy2, x, y, Q, Q, A, A, AT, AT)

    return (res, t1[0, 0], t2[0, 0], t3)
